# baseline (speedup 1.0000x reference)
import jax
import jax.numpy as jnp
from jax import lax
from jax.experimental import pallas as pl
from jax.experimental.pallas import tpu as pltpu
from jax.experimental.pallas import tpu_sc as plsc

NUM_CLASSES = 100000
EMBED_DIM = 64
BATCH = 16384

_NC = 2
_NS = 16
_NW = _NC * _NS
_RW = 1664
_NRANGE = 2 * _NW
_CAP = 512
_LO_DMA_MAX = 98432
_NFB = EMBED_DIM // 8


def _zero16(ref, base):
    ref[pl.ds(base, 16)] = jnp.zeros((16,), jnp.int32)


def _center_loss_body(tgt_hbm, emb_hbm, ctrT_hbm, out_hbm,
                      tv, tloc_a, ipos_a, tloc_b, ipos_b,
                      widx, erows, slab0, slab1, out_v,
                      gsem, ssem):
    wid = lax.axis_index("s") * _NC + lax.axis_index("c")
    lo0 = wid * (2 * _RW)

    pltpu.sync_copy(tgt_hbm, tv)

    iota16 = lax.iota(jnp.int32, 16)

    def scan_step(k, carry):
        off_a, off_b = carry
        t16 = tv[pl.ds(k * 16, 16)]
        i16 = iota16 + k * 16
        rel = t16 - lo0
        m_a = (rel >= 0) & (rel < _RW)
        m_b = (rel >= _RW) & (rel < 2 * _RW)
        plsc.store_compressed(tloc_a.at[pl.ds(off_a, 16)], rel, mask=m_a)
        plsc.store_compressed(ipos_a.at[pl.ds(off_a, 16)], i16, mask=m_a)
        plsc.store_compressed(tloc_b.at[pl.ds(off_b, 16)], rel - _RW, mask=m_b)
        plsc.store_compressed(ipos_b.at[pl.ds(off_b, 16)], i16, mask=m_b)
        n_a = plsc.all_reduce_population_count(m_a)[0]
        n_b = plsc.all_reduce_population_count(m_b)[0]
        return off_a + n_a, off_b + n_b

    n_a, n_b = lax.fori_loop(0, BATCH // 16, scan_step, (0, 0))

    _zero16(tloc_a, n_a)
    _zero16(ipos_a, n_a)
    _zero16(tloc_b, n_b)
    _zero16(ipos_b, n_b)

    zero = jnp.zeros((16,), jnp.float32)
    accs = (zero, zero, zero, zero)

    for rhalf in range(2):
        tloc = tloc_a if rhalf == 0 else tloc_b
        ipos = ipos_a if rhalf == 0 else ipos_b
        n = n_a if rhalf == 0 else n_b
        lo_sub = lo0 + rhalf * _RW
        lo_dma = jnp.minimum(lo_sub, _LO_DMA_MAX)
        lo_dma = pl.multiple_of(lo_dma, 128)
        delta16 = jnp.broadcast_to(lo_sub - lo_dma, (16,)).astype(jnp.int32)

        for j in range(4):
            for k in range(8):
                _zero16(widx.at[j], k * 16)

        def bld(k, _):
            v = ipos[pl.ds(k * 16, 16)] >> 1
            widx[k >> 3, pl.ds((k & 7) * 16, 16)] = v
            return 0

        lax.fori_loop(0, (n + 15) >> 4, bld, 0)

        gathers = []
        for j in range(4):
            gathers.append(pltpu.async_copy(
                emb_hbm.at[widx.at[j]],
                erows.at[pl.ds(j * 128, 128)],
                gsem))
        slab_cp = pltpu.async_copy(
            ctrT_hbm.at[pl.ds(0, 8), pl.ds(lo_dma, _RW)], slab0, ssem)
        for g in gathers:
            g.wait()

        ng = (n + 15) >> 4
        for fb in range(_NFB):
            slab_cur = slab0 if fb % 2 == 0 else slab1
            slab_nxt = slab1 if fb % 2 == 0 else slab0
            slab_cp.wait()
            if fb < _NFB - 1:
                slab_cp = pltpu.async_copy(
                    ctrT_hbm.at[pl.ds((fb + 1) * 8, 8), pl.ds(lo_dma, _RW)],
                    slab_nxt, ssem)

            def grp(g, a, slab_cur=slab_cur, fb=fb, tloc=tloc, ipos=ipos,
                    delta16=delta16):
                tl = tloc[pl.ds(g * 16, 16)] + delta16
                ip = ipos[pl.ds(g * 16, 16)]
                par = (ip & 1) << 6
                row = iota16 + g * 16
                valid = row < n
                fzero = jnp.zeros((16,), jnp.float32)
                new = list(a)
                for c in range(8):
                    c16 = jnp.full((16,), c, jnp.int32)
                    cv = plsc.load_gather(slab_cur, [c16, tl])
                    ev = plsc.load_gather(erows, [row, par + (8 * fb + c)])
                    d = jnp.where(valid, ev - cv, fzero)
                    new[c & 3] = new[c & 3] + d * d
                return tuple(new)

            accs = lax.fori_loop(0, ng, grp, accs)

    total = (accs[0] + accs[1]) + (accs[2] + accs[3])
    out_v[...] = total * jnp.float32(1.0 / BATCH)
    pltpu.sync_copy(out_v, out_hbm.at[wid])


@jax.jit
def _center_loss(embeddings, targets, centers):
    tgt = targets.astype(jnp.int32)
    emb2 = embeddings.reshape(BATCH // 2, 2 * EMBED_DIM)
    ctrT = centers.T
    mesh = plsc.VectorSubcoreMesh(core_axis_name="c", subcore_axis_name="s")
    partials = pl.kernel(
        _center_loss_body,
        mesh=mesh,
        out_type=jax.ShapeDtypeStruct((_NW, 16), jnp.float32),
        scratch_types=[
            pltpu.VMEM((BATCH,), jnp.int32),
            pltpu.VMEM((_CAP + 16,), jnp.int32),
            pltpu.VMEM((_CAP + 16,), jnp.int32),
            pltpu.VMEM((_CAP + 16,), jnp.int32),
            pltpu.VMEM((_CAP + 16,), jnp.int32),
            pltpu.VMEM((4, 128), jnp.int32),
            pltpu.VMEM((_CAP, 2 * EMBED_DIM), jnp.float32),
            pltpu.VMEM((8, _RW), jnp.float32),
            pltpu.VMEM((8, _RW), jnp.float32),
            pltpu.VMEM((16,), jnp.float32),
            pltpu.SemaphoreType.DMA,
            pltpu.SemaphoreType.DMA,
        ],
        compiler_params=pltpu.CompilerParams(
            use_tc_tiling_on_sc=True, needs_layout_passes=False),
    )(tgt, emb2, ctrT)
    return jnp.sum(partials)


def kernel(embeddings, targets, centers):
    return _center_loss(embeddings, targets, centers)


# speedup vs baseline: 9.2177x; 9.2177x over previous
import jax
import jax.numpy as jnp
from jax import lax
from jax.experimental import pallas as pl
from jax.experimental.pallas import tpu as pltpu
from jax.experimental.pallas import tpu_sc as plsc

NUM_CLASSES = 100000
EMBED_DIM = 64
BATCH = 16384

_NC = 2
_NS = 16
_NW = _NC * _NS
_RW = 1664
_NRANGE = 2 * _NW
_CAP = 512
_LO_DMA_MAX = 98432
_NFB = EMBED_DIM // 8


def _zero16(ref, base):
    ref[pl.ds(base, 16)] = jnp.zeros((16,), jnp.int32)


def _center_loss_body(tgt_hbm, emb_hbm, ctrT_hbm, out_hbm,
                      tv, tloc_a, ipos_a, tloc_b, ipos_b,
                      widx, erows, slab0, slab1, out_v,
                      gsem, ssem):
    wid = lax.axis_index("s") * _NC + lax.axis_index("c")
    lo0 = wid * (2 * _RW)

    pltpu.sync_copy(tgt_hbm, tv)

    iota16 = lax.iota(jnp.int32, 16)

    def scan_step(k, carry):
        off_a, off_b = carry
        t16 = tv[pl.ds(k * 16, 16)]
        i16 = iota16 + k * 16
        rel = t16 - lo0
        m_a = (rel >= 0) & (rel < _RW)
        m_b = (rel >= _RW) & (rel < 2 * _RW)
        plsc.store_compressed(tloc_a.at[pl.ds(off_a, 16)], rel, mask=m_a)
        plsc.store_compressed(ipos_a.at[pl.ds(off_a, 16)], i16, mask=m_a)
        plsc.store_compressed(tloc_b.at[pl.ds(off_b, 16)], rel - _RW, mask=m_b)
        plsc.store_compressed(ipos_b.at[pl.ds(off_b, 16)], i16, mask=m_b)
        n_a = plsc.all_reduce_population_count(m_a)[0]
        n_b = plsc.all_reduce_population_count(m_b)[0]
        return off_a + n_a, off_b + n_b

    n_a, n_b = lax.fori_loop(0, BATCH // 16, scan_step, (0, 0))

    _zero16(tloc_a, n_a)
    _zero16(ipos_a, n_a)
    _zero16(tloc_b, n_b)
    _zero16(ipos_b, n_b)

    zero = jnp.zeros((16,), jnp.float32)
    accs = (zero, zero, zero, zero)

    for rhalf in range(2):
        tloc = tloc_a if rhalf == 0 else tloc_b
        ipos = ipos_a if rhalf == 0 else ipos_b
        n = n_a if rhalf == 0 else n_b
        lo_sub = lo0 + rhalf * _RW
        lo_dma = jnp.minimum(lo_sub, _LO_DMA_MAX)
        lo_dma = pl.multiple_of(lo_dma, 128)
        delta16 = jnp.broadcast_to(lo_sub - lo_dma, (16,)).astype(jnp.int32)

        spread0 = (wid * 256) & (BATCH // 2 - 1)
        for j in range(4):
            for k in range(8):
                fill = (iota16 + (spread0 + (j * 8 + k) * 16)) & (BATCH // 2 - 1)
                widx[j, pl.ds(k * 16, 16)] = fill

        def bld(k, _):
            lane = iota16 + k * 16
            fill = (lane + spread0) & (BATCH // 2 - 1)
            v = jnp.where(lane < n, ipos[pl.ds(k * 16, 16)] >> 1, fill)
            widx[k >> 3, pl.ds((k & 7) * 16, 16)] = v
            return 0

        lax.fori_loop(0, (n + 15) >> 4, bld, 0)

        gathers = []
        for j in range(4):
            gathers.append(pltpu.async_copy(
                emb_hbm.at[widx.at[j]],
                erows.at[pl.ds(j * 128, 128)],
                gsem))
        slab_cp = pltpu.async_copy(
            ctrT_hbm.at[pl.ds(0, 8), pl.ds(lo_dma, _RW)], slab0, ssem)
        for g in gathers:
            g.wait()

        ng = (n + 15) >> 4
        for fb in range(_NFB):
            slab_cur = slab0 if fb % 2 == 0 else slab1
            slab_nxt = slab1 if fb % 2 == 0 else slab0
            slab_cp.wait()
            if fb < _NFB - 1:
                slab_cp = pltpu.async_copy(
                    ctrT_hbm.at[pl.ds((fb + 1) * 8, 8), pl.ds(lo_dma, _RW)],
                    slab_nxt, ssem)

            def grp(g, a, slab_cur=slab_cur, fb=fb, tloc=tloc, ipos=ipos,
                    delta16=delta16):
                tl = tloc[pl.ds(g * 16, 16)] + delta16
                ip = ipos[pl.ds(g * 16, 16)]
                par = (ip & 1) << 6
                row = iota16 + g * 16
                valid = row < n
                fzero = jnp.zeros((16,), jnp.float32)
                new = list(a)
                for c in range(8):
                    c16 = jnp.full((16,), c, jnp.int32)
                    cv = plsc.load_gather(slab_cur, [c16, tl])
                    ev = plsc.load_gather(erows, [row, par + (8 * fb + c)])
                    d = jnp.where(valid, ev - cv, fzero)
                    new[c & 3] = new[c & 3] + d * d
                return tuple(new)

            accs = lax.fori_loop(0, ng, grp, accs)

    total = (accs[0] + accs[1]) + (accs[2] + accs[3])
    out_v[...] = total * jnp.float32(1.0 / BATCH)
    pltpu.sync_copy(out_v, out_hbm.at[wid])


@jax.jit
def _center_loss(embeddings, targets, centers):
    tgt = targets.astype(jnp.int32)
    emb2 = embeddings.reshape(BATCH // 2, 2 * EMBED_DIM)
    ctrT = centers.T
    mesh = plsc.VectorSubcoreMesh(core_axis_name="c", subcore_axis_name="s")
    partials = pl.kernel(
        _center_loss_body,
        mesh=mesh,
        out_type=jax.ShapeDtypeStruct((_NW, 16), jnp.float32),
        scratch_types=[
            pltpu.VMEM((BATCH,), jnp.int32),
            pltpu.VMEM((_CAP + 16,), jnp.int32),
            pltpu.VMEM((_CAP + 16,), jnp.int32),
            pltpu.VMEM((_CAP + 16,), jnp.int32),
            pltpu.VMEM((_CAP + 16,), jnp.int32),
            pltpu.VMEM((4, 128), jnp.int32),
            pltpu.VMEM((_CAP, 2 * EMBED_DIM), jnp.float32),
            pltpu.VMEM((8, _RW), jnp.float32),
            pltpu.VMEM((8, _RW), jnp.float32),
            pltpu.VMEM((16,), jnp.float32),
            pltpu.SemaphoreType.DMA,
            pltpu.SemaphoreType.DMA,
        ],
        compiler_params=pltpu.CompilerParams(
            use_tc_tiling_on_sc=True, needs_layout_passes=False),
    )(tgt, emb2, ctrT)
    return jnp.sum(partials)


def kernel(embeddings, targets, centers):
    return _center_loss(embeddings, targets, centers)


# speedup vs baseline: 9.2203x; 1.0003x over previous
import jax
import jax.numpy as jnp
from jax import lax
from jax.experimental import pallas as pl
from jax.experimental.pallas import tpu as pltpu
from jax.experimental.pallas import tpu_sc as plsc

NUM_CLASSES = 100000
EMBED_DIM = 64
BATCH = 16384

_NC = 2
_NS = 16
_NW = _NC * _NS
_RW = 1664
_NRANGE = 2 * _NW
_CAP = 512
_LO_DMA_MAX = 98432
_NFB = EMBED_DIM // 8


def _zero16(ref, base):
    ref[pl.ds(base, 16)] = jnp.zeros((16,), jnp.int32)


def _center_loss_body(tgt_hbm, emb_hbm, ctrT_hbm, out_hbm,
                      tv, tloc_a, ipos_a, tloc_b, ipos_b,
                      widx, erows, slab0, slab1, out_v,
                      gsem, ssem):
    wid = lax.axis_index("s") * _NC + lax.axis_index("c")
    lo0 = wid * (2 * _RW)

    pltpu.sync_copy(tgt_hbm, tv)

    iota16 = lax.iota(jnp.int32, 16)

    def scan_step(k, carry):
        off_a, off_b = carry
        t16 = tv[pl.ds(k * 16, 16)]
        i16 = iota16 + k * 16
        rel = t16 - lo0
        m_a = (rel >= 0) & (rel < _RW)
        m_b = (rel >= _RW) & (rel < 2 * _RW)
        plsc.store_compressed(tloc_a.at[pl.ds(off_a, 16)], rel, mask=m_a)
        plsc.store_compressed(ipos_a.at[pl.ds(off_a, 16)], i16, mask=m_a)
        plsc.store_compressed(tloc_b.at[pl.ds(off_b, 16)], rel - _RW, mask=m_b)
        plsc.store_compressed(ipos_b.at[pl.ds(off_b, 16)], i16, mask=m_b)
        n_a = plsc.all_reduce_population_count(m_a)[0]
        n_b = plsc.all_reduce_population_count(m_b)[0]
        return off_a + n_a, off_b + n_b

    with jax.named_scope("scan_targets"):
        n_a, n_b = lax.fori_loop(0, BATCH // 16, scan_step, (0, 0))

    _zero16(tloc_a, n_a)
    _zero16(ipos_a, n_a)
    _zero16(tloc_b, n_b)
    _zero16(ipos_b, n_b)

    zero = jnp.zeros((16,), jnp.float32)
    accs = (zero, zero, zero, zero)

    for rhalf in range(2):
        tloc = tloc_a if rhalf == 0 else tloc_b
        ipos = ipos_a if rhalf == 0 else ipos_b
        n = n_a if rhalf == 0 else n_b
        lo_sub = lo0 + rhalf * _RW
        lo_dma = jnp.minimum(lo_sub, _LO_DMA_MAX)
        lo_dma = pl.multiple_of(lo_dma, 128)
        delta16 = jnp.broadcast_to(lo_sub - lo_dma, (16,)).astype(jnp.int32)

        spread0 = (wid * 256) & (BATCH // 2 - 1)
        for j in range(4):
            for k in range(8):
                fill = (iota16 + (spread0 + (j * 8 + k) * 16)) & (BATCH // 2 - 1)
                widx[j, pl.ds(k * 16, 16)] = fill

        def bld(k, _):
            lane = iota16 + k * 16
            fill = (lane + spread0) & (BATCH // 2 - 1)
            v = jnp.where(lane < n, ipos[pl.ds(k * 16, 16)] >> 1, fill)
            widx[k >> 3, pl.ds((k & 7) * 16, 16)] = v
            return 0

        lax.fori_loop(0, (n + 15) >> 4, bld, 0)

        with jax.named_scope("egather"):
            gathers = []
            for j in range(4):
                gathers.append(pltpu.async_copy(
                    emb_hbm.at[widx.at[j]],
                    erows.at[pl.ds(j * 128, 128)],
                    gsem))
            slab_cp = pltpu.async_copy(
                ctrT_hbm.at[pl.ds(0, 8), pl.ds(lo_dma, _RW)], slab0, ssem)
            for g in gathers:
                g.wait()

        ng = (n + 15) >> 4
        scope = jax.named_scope("fbcompute")
        scope.__enter__()
        for fb in range(_NFB):
            slab_cur = slab0 if fb % 2 == 0 else slab1
            slab_nxt = slab1 if fb % 2 == 0 else slab0
            slab_cp.wait()
            if fb < _NFB - 1:
                slab_cp = pltpu.async_copy(
                    ctrT_hbm.at[pl.ds((fb + 1) * 8, 8), pl.ds(lo_dma, _RW)],
                    slab_nxt, ssem)

            def grp(g, a, slab_cur=slab_cur, fb=fb, tloc=tloc, ipos=ipos,
                    delta16=delta16):
                tl = tloc[pl.ds(g * 16, 16)] + delta16
                ip = ipos[pl.ds(g * 16, 16)]
                par = (ip & 1) << 6
                row = iota16 + g * 16
                valid = row < n
                fzero = jnp.zeros((16,), jnp.float32)
                new = list(a)
                for c in range(8):
                    c16 = jnp.full((16,), c, jnp.int32)
                    cv = plsc.load_gather(slab_cur, [c16, tl])
                    ev = plsc.load_gather(erows, [row, par + (8 * fb + c)])
                    d = jnp.where(valid, ev - cv, fzero)
                    new[c & 3] = new[c & 3] + d * d
                return tuple(new)

            accs = lax.fori_loop(0, ng, grp, accs)
        scope.__exit__(None, None, None)

    total = (accs[0] + accs[1]) + (accs[2] + accs[3])
    out_v[...] = total * jnp.float32(1.0 / BATCH)
    pltpu.sync_copy(out_v, out_hbm.at[wid])


@jax.jit
def _center_loss(embeddings, targets, centers):
    tgt = targets.astype(jnp.int32)
    emb2 = embeddings.reshape(BATCH // 2, 2 * EMBED_DIM)
    ctrT = centers.T
    mesh = plsc.VectorSubcoreMesh(core_axis_name="c", subcore_axis_name="s")
    partials = pl.kernel(
        _center_loss_body,
        mesh=mesh,
        out_type=jax.ShapeDtypeStruct((_NW, 16), jnp.float32),
        scratch_types=[
            pltpu.VMEM((BATCH,), jnp.int32),
            pltpu.VMEM((_CAP + 16,), jnp.int32),
            pltpu.VMEM((_CAP + 16,), jnp.int32),
            pltpu.VMEM((_CAP + 16,), jnp.int32),
            pltpu.VMEM((_CAP + 16,), jnp.int32),
            pltpu.VMEM((4, 128), jnp.int32),
            pltpu.VMEM((_CAP, 2 * EMBED_DIM), jnp.float32),
            pltpu.VMEM((8, _RW), jnp.float32),
            pltpu.VMEM((8, _RW), jnp.float32),
            pltpu.VMEM((16,), jnp.float32),
            pltpu.SemaphoreType.DMA,
            pltpu.SemaphoreType.DMA,
        ],
        compiler_params=pltpu.CompilerParams(
            use_tc_tiling_on_sc=True, needs_layout_passes=False),
    )(tgt, emb2, ctrT)
    return jnp.sum(partials)


def kernel(embeddings, targets, centers):
    return _center_loss(embeddings, targets, centers)


# speedup vs baseline: 9.6125x; 1.0425x over previous
import jax
import jax.numpy as jnp
from jax import lax
from jax.experimental import pallas as pl
from jax.experimental.pallas import tpu as pltpu
from jax.experimental.pallas import tpu_sc as plsc

NUM_CLASSES = 100000
EMBED_DIM = 64
BATCH = 16384

_NC = 2
_NS = 16
_NW = _NC * _NS
_RW = 1664
_CAP = 416
_LO_DMA_MAX = 98432
_NFB = EMBED_DIM // 8
_NSLAB = 2 * _NFB
_HB = BATCH // 2 - 1


def _center_loss_body(tgt_hbm, emb_hbm, ctrT_hbm, out_hbm,
                      tv, plist_a, plist_b, widx, erows,
                      sl0, sl1, sl2, sl3, out_v, gsem, ssem):
    wid = lax.axis_index("s") * _NC + lax.axis_index("c")
    lo0 = wid * (2 * _RW)
    slabs = (sl0, sl1, sl2, sl3)

    lo_a = pl.multiple_of(jnp.minimum(lo0, _LO_DMA_MAX), 128)
    lo_b = pl.multiple_of(jnp.minimum(lo0 + _RW, _LO_DMA_MAX), 128)

    def slab_fire(g):
        fb = g % _NFB
        lo = lo_a if g < _NFB else lo_b
        return pltpu.async_copy(
            ctrT_hbm.at[pl.ds(fb * 8, 8), pl.ds(lo, _RW)],
            slabs[g % 4], ssem)

    pending = [slab_fire(g) for g in range(3)] + [None]

    iota16 = lax.iota(jnp.int32, 16)
    offs = (0, 0)
    for p in range(2):
        pltpu.sync_copy(tgt_hbm.at[pl.ds(p * (BATCH // 2), BATCH // 2)], tv)

        def scan_step(k, carry, p=p):
            off_a, off_b = carry
            t16 = tv[pl.ds(k * 16, 16)]
            i16 = iota16 + (k * 16 + p * (BATCH // 2))
            rel = t16 - lo0
            m_a = (rel >= 0) & (rel < _RW)
            m_b = (rel >= _RW) & (rel < 2 * _RW)
            pk_a = (rel << 14) | i16
            pk_b = pk_a - (_RW << 14)
            plsc.store_compressed(plist_a.at[pl.ds(off_a, 16)], pk_a, mask=m_a)
            plsc.store_compressed(plist_b.at[pl.ds(off_b, 16)], pk_b, mask=m_b)
            n_a = plsc.all_reduce_population_count(m_a)[0]
            n_b = plsc.all_reduce_population_count(m_b)[0]
            return off_a + n_a, off_b + n_b

        offs = lax.fori_loop(0, BATCH // 32, scan_step, offs)
    n_a, n_b = offs

    plist_a[pl.ds(n_a, 16)] = jnp.zeros((16,), jnp.int32)
    plist_b[pl.ds(n_b, 16)] = jnp.zeros((16,), jnp.int32)

    zero = jnp.zeros((16,), jnp.float32)
    fzero = zero
    accs = (zero, zero, zero, zero)
    spread0 = (wid * 256) & _HB

    def build_widx(plist, n):
        def bld(k, _):
            lane = iota16 + k * 16
            fill = (lane + spread0) & _HB
            off = jnp.minimum(k * 16, _CAP)
            v = jnp.where(lane < n,
                          (plist[pl.ds(off, 16)] & 0x3FFF) >> 1, fill)
            widx[k >> 3, pl.ds((k & 7) * 16, 16)] = v
            return 0
        lax.fori_loop(0, 32, bld, 0)

    def fire_egather(n):
        gathers = []
        for j in range(4):
            gathers.append(pltpu.async_copy(
                emb_hbm.at[widx.at[j]],
                erows.at[pl.ds(j * 128, 128)],
                gsem))
        for g in gathers:
            g.wait()

    for rhalf in range(2):
        plist = plist_a if rhalf == 0 else plist_b
        n = n_a if rhalf == 0 else n_b
        lo_sub = lo0 + rhalf * _RW
        lo_dma = lo_a if rhalf == 0 else lo_b
        delta16 = jnp.broadcast_to(lo_sub - lo_dma, (16,)).astype(jnp.int32)

        build_widx(plist, n)
        fire_egather(n)

        ng = (n + 15) >> 4
        for fb in range(_NFB):
            g = rhalf * _NFB + fb
            slab_cur = slabs[g % 4]
            pending[g % 4].wait()
            if g + 3 < _NSLAB:
                pending[(g + 3) % 4] = slab_fire(g + 3)

            def grp(gi, a, slab_cur=slab_cur, fb=fb, plist=plist,
                    delta16=delta16, n=n):
                pk = plist[pl.ds(gi * 16, 16)]
                tl = (pk >> 14) + delta16
                par = (pk & 1) << 6
                row = iota16 + gi * 16
                valid = row < n
                new = list(a)
                for c in range(8):
                    c16 = jnp.full((16,), c, jnp.int32)
                    cv = plsc.load_gather(slab_cur, [c16, tl])
                    ev = plsc.load_gather(erows, [row, par + (8 * fb + c)])
                    d = jnp.where(valid, ev - cv, fzero)
                    new[c & 3] = new[c & 3] + d * d
                return tuple(new)

            accs = lax.fori_loop(0, ng, grp, accs)

    total = (accs[0] + accs[1]) + (accs[2] + accs[3])
    out_v[...] = total * jnp.float32(1.0 / BATCH)
    pltpu.sync_copy(out_v, out_hbm.at[wid])


@jax.jit
def _center_loss(embeddings, targets, centers):
    tgt = targets.astype(jnp.int32)
    emb2 = embeddings.reshape(BATCH // 2, 2 * EMBED_DIM)
    ctrT = centers.T
    mesh = plsc.VectorSubcoreMesh(core_axis_name="c", subcore_axis_name="s")
    partials = pl.kernel(
        _center_loss_body,
        mesh=mesh,
        out_type=jax.ShapeDtypeStruct((_NW, 16), jnp.float32),
        scratch_types=[
            pltpu.VMEM((BATCH // 2,), jnp.int32),
            pltpu.VMEM((_CAP + 16,), jnp.int32),
            pltpu.VMEM((_CAP + 16,), jnp.int32),
            pltpu.VMEM((4, 128), jnp.int32),
            pltpu.VMEM((_CAP + 96, 2 * EMBED_DIM), jnp.float32),
            pltpu.VMEM((8, _RW), jnp.float32),
            pltpu.VMEM((8, _RW), jnp.float32),
            pltpu.VMEM((8, _RW), jnp.float32),
            pltpu.VMEM((8, _RW), jnp.float32),
            pltpu.VMEM((16,), jnp.float32),
            pltpu.SemaphoreType.DMA,
            pltpu.SemaphoreType.DMA,
        ],
        compiler_params=pltpu.CompilerParams(
            use_tc_tiling_on_sc=True, needs_layout_passes=False),
    )(tgt, emb2, ctrT)
    return jnp.sum(partials)


def kernel(embeddings, targets, centers):
    return _center_loss(embeddings, targets, centers)


# speedup vs baseline: 9.8036x; 1.0199x over previous
import jax
import jax.numpy as jnp
from jax import lax
from jax.experimental import pallas as pl
from jax.experimental.pallas import tpu as pltpu
from jax.experimental.pallas import tpu_sc as plsc

NUM_CLASSES = 100000
EMBED_DIM = 64
BATCH = 16384

_NC = 2
_NS = 16
_NW = _NC * _NS
_RW = 1664
_CAP = 416
_LO_DMA_MAX = 98432
_NFB = EMBED_DIM // 8
_NSLAB = 2 * _NFB
_HB = BATCH // 2 - 1


def _center_loss_body(tgt_hbm, emb_hbm, ctrT_hbm, out_hbm,
                      tv, plist_a, plist_b, widx, erows,
                      sl0, sl1, sl2, sl3, out_v, gsem, ssem):
    wid = lax.axis_index("s") * _NC + lax.axis_index("c")
    lo0 = wid * (2 * _RW)
    slabs = (sl0, sl1, sl2, sl3)

    lo_a = pl.multiple_of(jnp.minimum(lo0, _LO_DMA_MAX), 128)
    lo_b = pl.multiple_of(jnp.minimum(lo0 + _RW, _LO_DMA_MAX), 128)

    def slab_fire(g):
        fb = g % _NFB
        lo = lo_a if g < _NFB else lo_b
        return pltpu.async_copy(
            ctrT_hbm.at[pl.ds(fb * 8, 8), pl.ds(lo, _RW)],
            slabs[g % 4], ssem)

    pending = [slab_fire(g) for g in range(3)] + [None]

    iota16 = lax.iota(jnp.int32, 16)
    offs = (0, 0)
    for p in range(2):
        pltpu.sync_copy(tgt_hbm.at[pl.ds(p * (BATCH // 2), BATCH // 2)], tv)

        def scan_step(k, carry, p=p):
            off_a, off_b = carry
            t16 = tv[pl.ds(k * 16, 16)]
            i16 = iota16 + (k * 16 + p * (BATCH // 2))
            rel = t16 - lo0
            m_a = (rel >= 0) & (rel < _RW)
            m_b = (rel >= _RW) & (rel < 2 * _RW)
            pk_a = (rel << 14) | i16
            pk_b = pk_a - (_RW << 14)
            plsc.store_compressed(plist_a.at[pl.ds(off_a, 16)], pk_a, mask=m_a)
            plsc.store_compressed(plist_b.at[pl.ds(off_b, 16)], pk_b, mask=m_b)
            n_a = plsc.all_reduce_population_count(m_a)[0]
            n_b = plsc.all_reduce_population_count(m_b)[0]
            return off_a + n_a, off_b + n_b

        offs = lax.fori_loop(0, BATCH // 32, scan_step, offs)
    n_a, n_b = offs

    plist_a[pl.ds(n_a, 16)] = jnp.zeros((16,), jnp.int32)
    plist_b[pl.ds(n_b, 16)] = jnp.zeros((16,), jnp.int32)

    zero = jnp.zeros((16,), jnp.float32)
    fzero = zero
    accs = (zero, zero, zero, zero)
    spread0 = (wid * 256) & _HB

    def build_widx(plist, n):
        def bld(k, _):
            lane = iota16 + k * 16
            fill = (lane + spread0) & _HB
            off = jnp.minimum(k * 16, _CAP)
            v = jnp.where(lane < n,
                          (plist[pl.ds(off, 16)] & 0x3FFF) >> 1, fill)
            widx[k >> 3, pl.ds((k & 7) * 16, 16)] = v
            return 0
        lax.fori_loop(0, 32, bld, 0)

    def fire_egather(n):
        first = pltpu.async_copy(
            emb_hbm.at[widx.at[0]], erows.at[pl.ds(0, 128)], gsem)
        for j in range(1, 4):
            @pl.when(j * 128 < n)
            def _(j=j):
                pltpu.async_copy(
                    emb_hbm.at[widx.at[j]],
                    erows.at[pl.ds(j * 128, 128)], gsem)
        first.wait()
        for j in range(1, 4):
            @pl.when(j * 128 < n)
            def _(j=j):
                pltpu.make_async_copy(
                    emb_hbm.at[widx.at[j]],
                    erows.at[pl.ds(j * 128, 128)], gsem).wait()

    for rhalf in range(2):
        plist = plist_a if rhalf == 0 else plist_b
        n = n_a if rhalf == 0 else n_b
        lo_sub = lo0 + rhalf * _RW
        lo_dma = lo_a if rhalf == 0 else lo_b
        delta16 = jnp.broadcast_to(lo_sub - lo_dma, (16,)).astype(jnp.int32)

        build_widx(plist, n)
        fire_egather(n)

        ng = (n + 15) >> 4
        for fb in range(_NFB):
            g = rhalf * _NFB + fb
            slab_cur = slabs[g % 4]
            pending[g % 4].wait()
            if g + 3 < _NSLAB:
                pending[(g + 3) % 4] = slab_fire(g + 3)

            def grp(gi, a, slab_cur=slab_cur, fb=fb, plist=plist,
                    delta16=delta16, n=n):
                pk = plist[pl.ds(gi * 16, 16)]
                tl = (pk >> 14) + delta16
                par = (pk & 1) << 6
                row = iota16 + gi * 16
                valid = row < n
                new = list(a)
                for c in range(8):
                    c16 = jnp.full((16,), c, jnp.int32)
                    cv = plsc.load_gather(slab_cur, [c16, tl])
                    ev = plsc.load_gather(erows, [row, par + (8 * fb + c)])
                    d = jnp.where(valid, ev - cv, fzero)
                    new[c & 3] = new[c & 3] + d * d
                return tuple(new)

            accs = lax.fori_loop(0, ng, grp, accs)

    total = (accs[0] + accs[1]) + (accs[2] + accs[3])
    out_v[...] = total * jnp.float32(1.0 / BATCH)
    pltpu.sync_copy(out_v, out_hbm.at[wid])


@jax.jit
def _center_loss(embeddings, targets, centers):
    tgt = targets.astype(jnp.int32)
    emb2 = embeddings.reshape(BATCH // 2, 2 * EMBED_DIM)
    ctrT = centers.T
    mesh = plsc.VectorSubcoreMesh(core_axis_name="c", subcore_axis_name="s")
    partials = pl.kernel(
        _center_loss_body,
        mesh=mesh,
        out_type=jax.ShapeDtypeStruct((_NW, 16), jnp.float32),
        scratch_types=[
            pltpu.VMEM((BATCH // 2,), jnp.int32),
            pltpu.VMEM((_CAP + 16,), jnp.int32),
            pltpu.VMEM((_CAP + 16,), jnp.int32),
            pltpu.VMEM((4, 128), jnp.int32),
            pltpu.VMEM((_CAP + 96, 2 * EMBED_DIM), jnp.float32),
            pltpu.VMEM((8, _RW), jnp.float32),
            pltpu.VMEM((8, _RW), jnp.float32),
            pltpu.VMEM((8, _RW), jnp.float32),
            pltpu.VMEM((8, _RW), jnp.float32),
            pltpu.VMEM((16,), jnp.float32),
            pltpu.SemaphoreType.DMA,
            pltpu.SemaphoreType.DMA,
        ],
        compiler_params=pltpu.CompilerParams(
            use_tc_tiling_on_sc=True, needs_layout_passes=False),
    )(tgt, emb2, ctrT)
    return jnp.sum(partials)


def kernel(embeddings, targets, centers):
    return _center_loss(embeddings, targets, centers)
